# Initial kernel scaffold; baseline (speedup 1.0000x reference)
import jax
import jax.numpy as jnp
from jax.experimental import pallas as pl


def kernel(x, edge_index, edge_attr, Wq, Wk, Wv, We, O_h_W, O_h_b, O_e_W, O_e_b, g1h, b1h, g1e, b1e, Fh1_W, Fh1_b, Fh2_W, Fh2_b, Fe1_W, Fe1_b, Fe2_W, Fe2_b, g2h, b2h, g2e, b2e):
    raise NotImplementedError("write your pallas kernel here")


# speedup vs baseline: 9.3351x; 9.3351x over previous
import functools

import jax
import jax.numpy as jnp
from jax import lax
from jax.experimental import pallas as pl
from jax.experimental.pallas import tpu as pltpu
from jax.experimental.pallas import tpu_sc as plsc

H = 8
C = 16
EPS = 1e-5
NC = 2
NS = 16
NW = NC * NS


def _headsum_mat(dtype=jnp.float32):
    r = lax.broadcasted_iota(jnp.int32, (128, 8), 0)
    c = lax.broadcasted_iota(jnp.int32, (128, 8), 1)
    return (r // C == c).astype(dtype)


def _headexp_mat(dtype=jnp.float32):
    r = lax.broadcasted_iota(jnp.int32, (8, 128), 0)
    c = lax.broadcasted_iota(jnp.int32, (8, 128), 1)
    return (c // C == r).astype(dtype)


CH = 128


def _sc_gather(x, src, dst):
    N, D = x.shape
    E = src.shape[0]
    nchunk = E // CH
    nfull = (nchunk // NW) * NW
    nit = nchunk // NW
    nleft = nchunk - nfull
    mesh = plsc.VectorSubcoreMesh(core_axis_name="c", subcore_axis_name="s")

    @functools.partial(
        pl.kernel,
        out_type=(jax.ShapeDtypeStruct((E, D), jnp.float32),
                  jax.ShapeDtypeStruct((E, D), jnp.float32)),
        mesh=mesh,
        scratch_types=[pltpu.VMEM((CH,), jnp.int32),
                       pltpu.VMEM((CH, D), jnp.float32)],
    )
    def k(x_hbm, src_hbm, dst_hbm, xs_hbm, xd_hbm, idx_v, rows_v):
        wid = lax.axis_index("s") * NC + lax.axis_index("c")

        def one(base):
            pltpu.sync_copy(src_hbm.at[pl.ds(base, CH)], idx_v)
            pltpu.sync_copy(x_hbm.at[idx_v], rows_v)
            pltpu.sync_copy(rows_v, xs_hbm.at[pl.ds(base, CH)])
            pltpu.sync_copy(dst_hbm.at[pl.ds(base, CH)], idx_v)
            pltpu.sync_copy(x_hbm.at[idx_v], rows_v)
            pltpu.sync_copy(rows_v, xd_hbm.at[pl.ds(base, CH)])

        @pl.loop(0, nit)
        def _(j):
            one((j * NW + wid) * CH)

        if nleft:
            @pl.when(wid < nleft)
            def _():
                one((nfull + wid) * CH)

    return k(x, src, dst)


def _tc_segsum(cu, dst1, N):
    E = cu.shape[0]
    T = 2048
    G = E // T

    def body(dst_s, cu_ref, nd_ref, acc):
        i = pl.program_id(0)

        @pl.when(i == 0)
        def _():
            acc[...] = jnp.zeros_like(acc)

        def one(k, _):
            d = dst_s[0, k]
            acc[pl.ds(d, 1), :] += cu_ref[pl.ds(k, 1), :]
            return 0

        lax.fori_loop(0, T, one, 0, unroll=8)

        @pl.when(i == G - 1)
        def _():
            nd_ref[...] = acc[...]

    return pl.pallas_call(
        body,
        grid=(G,),
        in_specs=[pl.BlockSpec((1, T), lambda i: (0, i),
                               memory_space=pltpu.SMEM),
                  pl.BlockSpec((T, 136), lambda i: (i, 0))],
        out_specs=pl.BlockSpec((N, 136), lambda i: (0, 0)),
        out_shape=jax.ShapeDtypeStruct((N, 136), jnp.float32),
        scratch_shapes=[pltpu.VMEM((N, 136), jnp.float32)],
        name="segsum",
    )(dst1, cu)


def _edge_attn(ea, xs, xd, Wq, Wk, Wv, We, OeW, Oeb):
    E, D = ea.shape
    T = 512
    G = E // T

    def body(ea_ref, xs_ref, xd_ref, wq, wk, wv, we, ow, ob,
             cu_ref, y1_ref, st_ref):
        i = pl.program_id(0)
        eav = ea_ref[...]
        xsv = xs_ref[...]
        P = lax.Precision.HIGHEST
        qd = jnp.dot(xd_ref[...], wq[...], precision=P)
        ks = jnp.dot(xsv, wk[...], precision=P)
        vs = jnp.dot(xsv, wv[...], precision=P)
        ep = jnp.dot(eav, we[...], precision=P)
        kj = ks + ep
        vj = vs + ep
        sf = qd * kj * 0.25
        score = jnp.dot(sf, _headsum_mat(), precision=lax.Precision.HIGHEST)
        ex = jnp.exp(score)
        exb = jnp.dot(ex, _headexp_mat(), precision=lax.Precision.HIGHEST)
        contrib = exb * vj
        cu_ref[...] = jnp.concatenate([contrib, ex], axis=1)
        y1 = eav + jnp.dot(sf, ow[...], precision=P) + ob[...]
        y1_ref[...] = y1
        s1 = jnp.sum(y1, axis=0, keepdims=True)
        s2 = jnp.sum(y1 * y1, axis=0, keepdims=True)
        upd = jnp.concatenate([s1, s2, jnp.zeros((6, 128), jnp.float32)],
                              axis=0)

        @pl.when(i == 0)
        def _():
            st_ref[...] = jnp.zeros_like(st_ref)

        st_ref[...] += upd

    full = pl.BlockSpec((128, 128), lambda i: (0, 0))
    row = pl.BlockSpec((1, 128), lambda i: (0, 0))
    tile = pl.BlockSpec((T, 128), lambda i: (i, 0))
    return pl.pallas_call(
        body,
        grid=(G,),
        in_specs=[tile, tile, tile, full, full, full, full, full, row],
        out_specs=[pl.BlockSpec((T, 136), lambda i: (i, 0)), tile,
                   pl.BlockSpec((8, 128), lambda i: (0, 0))],
        out_shape=(jax.ShapeDtypeStruct((E, 136), jnp.float32),
                   jax.ShapeDtypeStruct((E, D), jnp.float32),
                   jax.ShapeDtypeStruct((8, 128), jnp.float32)),
        name="edge_attn",
    )(ea, xs, xd, Wq, Wk, Wv, We, OeW, Oeb)


def _h_proj(x, nd, OhW, Ohb):
    N, D = x.shape
    T = 1000
    G = N // T

    def body(x_ref, nd_ref, ow, ob, h0_ref, st_ref):
        i = pl.program_id(0)
        nd = nd_ref[...]
        num = nd[:, 0:128]
        denb = jnp.dot(nd[:, 128:136], _headexp_mat(),
                       precision=lax.Precision.HIGHEST) + 1e-16
        hagg = num / denb
        h0 = jnp.dot(hagg, ow[...], precision=lax.Precision.HIGHEST) + ob[...] + x_ref[...]
        h0_ref[...] = h0
        s1 = jnp.sum(h0, axis=0, keepdims=True)
        s2 = jnp.sum(h0 * h0, axis=0, keepdims=True)
        upd = jnp.concatenate([s1, s2, jnp.zeros((6, 128), jnp.float32)],
                              axis=0)

        @pl.when(i == 0)
        def _():
            st_ref[...] = jnp.zeros_like(st_ref)

        st_ref[...] += upd

    return pl.pallas_call(
        body,
        grid=(G,),
        in_specs=[pl.BlockSpec((T, 128), lambda i: (i, 0)),
                  pl.BlockSpec((T, 136), lambda i: (i, 0)),
                  pl.BlockSpec((128, 128), lambda i: (0, 0)),
                  pl.BlockSpec((1, 128), lambda i: (0, 0))],
        out_specs=[pl.BlockSpec((T, 128), lambda i: (i, 0)),
                   pl.BlockSpec((8, 128), lambda i: (0, 0))],
        out_shape=(jax.ShapeDtypeStruct((N, D), jnp.float32),
                   jax.ShapeDtypeStruct((8, 128), jnp.float32)),
        name="h_proj",
    )(x, nd, OhW, Ohb)


def _ffn(t, st, g, b, F1W, F1b, F2W, F2b, rows, T):
    R, D = t.shape
    G = R // T
    inv = 1.0 / float(rows)

    def body(t_ref, sti_ref, gr, br, f1, f1b, f2, f2b, z_ref, st2_ref):
        i = pl.program_id(0)
        s = sti_ref[...]
        mu = s[0:1] * inv
        var = s[1:2] * inv - mu * mu
        rs = lax.rsqrt(var + EPS)
        tn = (t_ref[...] - mu) * rs * gr[...] + br[...]
        P = lax.Precision.HIGHEST
        hid = jnp.maximum(jnp.dot(tn, f1[...], precision=P) + f1b[...], 0.0)
        z = tn + jnp.dot(hid, f2[...], precision=P) + f2b[...]
        z_ref[...] = z
        s1 = jnp.sum(z, axis=0, keepdims=True)
        s2 = jnp.sum(z * z, axis=0, keepdims=True)
        upd = jnp.concatenate([s1, s2, jnp.zeros((6, 128), jnp.float32)],
                              axis=0)

        @pl.when(i == 0)
        def _():
            st2_ref[...] = jnp.zeros_like(st2_ref)

        st2_ref[...] += upd

    return pl.pallas_call(
        body,
        grid=(G,),
        in_specs=[pl.BlockSpec((T, 128), lambda i: (i, 0)),
                  pl.BlockSpec((8, 128), lambda i: (0, 0)),
                  pl.BlockSpec((1, 128), lambda i: (0, 0)),
                  pl.BlockSpec((1, 128), lambda i: (0, 0)),
                  pl.BlockSpec((128, 256), lambda i: (0, 0)),
                  pl.BlockSpec((1, 256), lambda i: (0, 0)),
                  pl.BlockSpec((256, 128), lambda i: (0, 0)),
                  pl.BlockSpec((1, 128), lambda i: (0, 0))],
        out_specs=[pl.BlockSpec((T, 128), lambda i: (i, 0)),
                   pl.BlockSpec((8, 128), lambda i: (0, 0))],
        out_shape=(jax.ShapeDtypeStruct((R, D), jnp.float32),
                   jax.ShapeDtypeStruct((8, 128), jnp.float32)),
        name="bn_ffn",
    )(t, st, g, b, F1W, F1b, F2W, F2b)


def _norm(z, st, g, b, rows, T):
    R, D = z.shape
    G = R // T
    inv = 1.0 / float(rows)

    def body(z_ref, sti_ref, gr, br, o_ref):
        s = sti_ref[...]
        mu = s[0:1] * inv
        var = s[1:2] * inv - mu * mu
        rs = lax.rsqrt(var + EPS)
        o_ref[...] = (z_ref[...] - mu) * rs * gr[...] + br[...]

    return pl.pallas_call(
        body,
        grid=(G,),
        in_specs=[pl.BlockSpec((T, 128), lambda i: (i, 0)),
                  pl.BlockSpec((8, 128), lambda i: (0, 0)),
                  pl.BlockSpec((1, 128), lambda i: (0, 0)),
                  pl.BlockSpec((1, 128), lambda i: (0, 0))],
        out_specs=pl.BlockSpec((T, 128), lambda i: (i, 0)),
        out_shape=jax.ShapeDtypeStruct((R, D), jnp.float32),
        name="bn_out",
    )(z, st, g, b)


def kernel(x, edge_index, edge_attr, Wq, Wk, Wv, We, O_h_W, O_h_b, O_e_W,
           O_e_b, g1h, b1h, g1e, b1e, Fh1_W, Fh1_b, Fh2_W, Fh2_b,
           Fe1_W, Fe1_b, Fe2_W, Fe2_b, g2h, b2h, g2e, b2e):
    N, D = x.shape
    E = edge_attr.shape[0]
    r = lambda v: v.reshape(1, -1)

    src = edge_index[0]
    dst = edge_index[1]
    xs, xd = _sc_gather(x, src, dst)
    cu, y1, st1e = _edge_attn(
        edge_attr, xs, xd, Wq, Wk, Wv, We, O_e_W, r(O_e_b))
    nd = _tc_segsum(cu, dst.reshape(1, E), N)
    h0, st1h = _h_proj(x, nd, O_h_W, r(O_h_b))
    zh, st2h = _ffn(h0, st1h, r(g1h), r(b1h), Fh1_W, r(Fh1_b), Fh2_W,
                    r(Fh2_b), rows=N, T=1000)
    h = _norm(zh, st2h, r(g2h), r(b2h), rows=N, T=1000)
    ze, st2e = _ffn(y1, st1e, r(g1e), r(b1e), Fe1_W, r(Fe1_b), Fe2_W,
                    r(Fe2_b), rows=E, T=512)
    e = _norm(ze, st2e, r(g2e), r(b2e), rows=E, T=512)
    return h, e


# speedup vs baseline: 10.0872x; 1.0806x over previous
import functools

import jax
import jax.numpy as jnp
from jax import lax
from jax.experimental import pallas as pl
from jax.experimental.pallas import tpu as pltpu
from jax.experimental.pallas import tpu_sc as plsc

H = 8
C = 16
EPS = 1e-5
NC = 2
NS = 16
NW = NC * NS


def _headsum_mat(dtype=jnp.float32):
    r = lax.broadcasted_iota(jnp.int32, (128, 8), 0)
    c = lax.broadcasted_iota(jnp.int32, (128, 8), 1)
    return (r // C == c).astype(dtype)


def _headexp_mat(dtype=jnp.float32):
    r = lax.broadcasted_iota(jnp.int32, (8, 128), 0)
    c = lax.broadcasted_iota(jnp.int32, (8, 128), 1)
    return (c // C == r).astype(dtype)


CH = 128


def _sc_gather(x, src, dst):
    N, D = x.shape
    E = src.shape[0]
    nchunk = E // CH
    nfull = (nchunk // NW) * NW
    nit = nchunk // NW
    nleft = nchunk - nfull
    mesh = plsc.VectorSubcoreMesh(core_axis_name="c", subcore_axis_name="s")

    @functools.partial(
        pl.kernel,
        out_type=(jax.ShapeDtypeStruct((E, D), jnp.float32),
                  jax.ShapeDtypeStruct((E, D), jnp.float32)),
        mesh=mesh,
        scratch_types=[pltpu.VMEM((CH,), jnp.int32),
                       pltpu.VMEM((CH, D), jnp.float32)],
    )
    def k(x_hbm, src_hbm, dst_hbm, xs_hbm, xd_hbm, idx_v, rows_v):
        wid = lax.axis_index("s") * NC + lax.axis_index("c")

        def one(base):
            pltpu.sync_copy(src_hbm.at[pl.ds(base, CH)], idx_v)
            pltpu.sync_copy(x_hbm.at[idx_v], rows_v)
            pltpu.sync_copy(rows_v, xs_hbm.at[pl.ds(base, CH)])
            pltpu.sync_copy(dst_hbm.at[pl.ds(base, CH)], idx_v)
            pltpu.sync_copy(x_hbm.at[idx_v], rows_v)
            pltpu.sync_copy(rows_v, xd_hbm.at[pl.ds(base, CH)])

        @pl.loop(0, nit)
        def _(j):
            one((j * NW + wid) * CH)

        if nleft:
            @pl.when(wid < nleft)
            def _():
                one((nfull + wid) * CH)

    return k(x, src, dst)


def _tc_segsum(cu, dst1, N):
    E = cu.shape[0]
    T = 2048
    G = E // T

    def body(dst_s, cu_ref, nd_ref, a0, a1, a2, a3):
        i = pl.program_id(0)
        banks = (a0, a1, a2, a3)

        @pl.when(i == 0)
        def _():
            for a in banks:
                a[...] = jnp.zeros_like(a)

        def one(k, _):
            for b, a in enumerate(banks):
                d = dst_s[0, k * 4 + b]
                a[pl.ds(d, 1), :] += cu_ref[pl.ds(k * 4 + b, 1), :]
            return 0

        lax.fori_loop(0, T // 4, one, 0, unroll=4)

        @pl.when(i == G - 1)
        def _():
            nd_ref[...] = (a0[...] + a1[...]) + (a2[...] + a3[...])

    return pl.pallas_call(
        body,
        grid=(G,),
        in_specs=[pl.BlockSpec((1, T), lambda i: (0, i),
                               memory_space=pltpu.SMEM),
                  pl.BlockSpec((T, 136), lambda i: (i, 0))],
        out_specs=pl.BlockSpec((N, 136), lambda i: (0, 0)),
        out_shape=jax.ShapeDtypeStruct((N, 136), jnp.float32),
        scratch_shapes=[pltpu.VMEM((N, 136), jnp.float32)] * 4,
        name="segsum",
    )(dst1, cu)


def _edge_attn(ea, xs, xd, Wq, Wk, Wv, We, OeW, Oeb):
    E, D = ea.shape
    T = 512
    G = E // T

    def body(ea_ref, xs_ref, xd_ref, wq, wk, wv, we, ow, ob,
             cu_ref, y1_ref, st_ref):
        i = pl.program_id(0)
        eav = ea_ref[...]
        xsv = xs_ref[...]
        P = lax.Precision.HIGHEST
        qd = jnp.dot(xd_ref[...], wq[...], precision=P)
        ks = jnp.dot(xsv, wk[...], precision=P)
        vs = jnp.dot(xsv, wv[...], precision=P)
        ep = jnp.dot(eav, we[...], precision=P)
        kj = ks + ep
        vj = vs + ep
        sf = qd * kj * 0.25
        score = jnp.dot(sf, _headsum_mat(), precision=lax.Precision.HIGHEST)
        ex = jnp.exp(score)
        exb = jnp.dot(ex, _headexp_mat(), precision=lax.Precision.HIGHEST)
        contrib = exb * vj
        cu_ref[...] = jnp.concatenate([contrib, ex], axis=1)
        y1 = eav + jnp.dot(sf, ow[...], precision=P) + ob[...]
        y1_ref[...] = y1
        s1 = jnp.sum(y1, axis=0, keepdims=True)
        s2 = jnp.sum(y1 * y1, axis=0, keepdims=True)
        upd = jnp.concatenate([s1, s2, jnp.zeros((6, 128), jnp.float32)],
                              axis=0)

        @pl.when(i == 0)
        def _():
            st_ref[...] = jnp.zeros_like(st_ref)

        st_ref[...] += upd

    full = pl.BlockSpec((128, 128), lambda i: (0, 0))
    row = pl.BlockSpec((1, 128), lambda i: (0, 0))
    tile = pl.BlockSpec((T, 128), lambda i: (i, 0))
    return pl.pallas_call(
        body,
        grid=(G,),
        in_specs=[tile, tile, tile, full, full, full, full, full, row],
        out_specs=[pl.BlockSpec((T, 136), lambda i: (i, 0)), tile,
                   pl.BlockSpec((8, 128), lambda i: (0, 0))],
        out_shape=(jax.ShapeDtypeStruct((E, 136), jnp.float32),
                   jax.ShapeDtypeStruct((E, D), jnp.float32),
                   jax.ShapeDtypeStruct((8, 128), jnp.float32)),
        name="edge_attn",
    )(ea, xs, xd, Wq, Wk, Wv, We, OeW, Oeb)


def _h_proj(x, nd, OhW, Ohb):
    N, D = x.shape
    T = 1000
    G = N // T

    def body(x_ref, nd_ref, ow, ob, h0_ref, st_ref):
        i = pl.program_id(0)
        nd = nd_ref[...]
        num = nd[:, 0:128]
        denb = jnp.dot(nd[:, 128:136], _headexp_mat(),
                       precision=lax.Precision.HIGHEST) + 1e-16
        hagg = num / denb
        h0 = jnp.dot(hagg, ow[...], precision=lax.Precision.HIGHEST) + ob[...] + x_ref[...]
        h0_ref[...] = h0
        s1 = jnp.sum(h0, axis=0, keepdims=True)
        s2 = jnp.sum(h0 * h0, axis=0, keepdims=True)
        upd = jnp.concatenate([s1, s2, jnp.zeros((6, 128), jnp.float32)],
                              axis=0)

        @pl.when(i == 0)
        def _():
            st_ref[...] = jnp.zeros_like(st_ref)

        st_ref[...] += upd

    return pl.pallas_call(
        body,
        grid=(G,),
        in_specs=[pl.BlockSpec((T, 128), lambda i: (i, 0)),
                  pl.BlockSpec((T, 136), lambda i: (i, 0)),
                  pl.BlockSpec((128, 128), lambda i: (0, 0)),
                  pl.BlockSpec((1, 128), lambda i: (0, 0))],
        out_specs=[pl.BlockSpec((T, 128), lambda i: (i, 0)),
                   pl.BlockSpec((8, 128), lambda i: (0, 0))],
        out_shape=(jax.ShapeDtypeStruct((N, D), jnp.float32),
                   jax.ShapeDtypeStruct((8, 128), jnp.float32)),
        name="h_proj",
    )(x, nd, OhW, Ohb)


def _ffn(t, st, g, b, F1W, F1b, F2W, F2b, rows, T):
    R, D = t.shape
    G = R // T
    inv = 1.0 / float(rows)

    def body(t_ref, sti_ref, gr, br, f1, f1b, f2, f2b, z_ref, st2_ref):
        i = pl.program_id(0)
        s = sti_ref[...]
        mu = s[0:1] * inv
        var = s[1:2] * inv - mu * mu
        rs = lax.rsqrt(var + EPS)
        tn = (t_ref[...] - mu) * rs * gr[...] + br[...]
        P = lax.Precision.HIGHEST
        hid = jnp.maximum(jnp.dot(tn, f1[...], precision=P) + f1b[...], 0.0)
        z = tn + jnp.dot(hid, f2[...], precision=P) + f2b[...]
        z_ref[...] = z
        s1 = jnp.sum(z, axis=0, keepdims=True)
        s2 = jnp.sum(z * z, axis=0, keepdims=True)
        upd = jnp.concatenate([s1, s2, jnp.zeros((6, 128), jnp.float32)],
                              axis=0)

        @pl.when(i == 0)
        def _():
            st2_ref[...] = jnp.zeros_like(st2_ref)

        st2_ref[...] += upd

    return pl.pallas_call(
        body,
        grid=(G,),
        in_specs=[pl.BlockSpec((T, 128), lambda i: (i, 0)),
                  pl.BlockSpec((8, 128), lambda i: (0, 0)),
                  pl.BlockSpec((1, 128), lambda i: (0, 0)),
                  pl.BlockSpec((1, 128), lambda i: (0, 0)),
                  pl.BlockSpec((128, 256), lambda i: (0, 0)),
                  pl.BlockSpec((1, 256), lambda i: (0, 0)),
                  pl.BlockSpec((256, 128), lambda i: (0, 0)),
                  pl.BlockSpec((1, 128), lambda i: (0, 0))],
        out_specs=[pl.BlockSpec((T, 128), lambda i: (i, 0)),
                   pl.BlockSpec((8, 128), lambda i: (0, 0))],
        out_shape=(jax.ShapeDtypeStruct((R, D), jnp.float32),
                   jax.ShapeDtypeStruct((8, 128), jnp.float32)),
        name="bn_ffn",
    )(t, st, g, b, F1W, F1b, F2W, F2b)


def _norm(z, st, g, b, rows, T):
    R, D = z.shape
    G = R // T
    inv = 1.0 / float(rows)

    def body(z_ref, sti_ref, gr, br, o_ref):
        s = sti_ref[...]
        mu = s[0:1] * inv
        var = s[1:2] * inv - mu * mu
        rs = lax.rsqrt(var + EPS)
        o_ref[...] = (z_ref[...] - mu) * rs * gr[...] + br[...]

    return pl.pallas_call(
        body,
        grid=(G,),
        in_specs=[pl.BlockSpec((T, 128), lambda i: (i, 0)),
                  pl.BlockSpec((8, 128), lambda i: (0, 0)),
                  pl.BlockSpec((1, 128), lambda i: (0, 0)),
                  pl.BlockSpec((1, 128), lambda i: (0, 0))],
        out_specs=pl.BlockSpec((T, 128), lambda i: (i, 0)),
        out_shape=jax.ShapeDtypeStruct((R, D), jnp.float32),
        name="bn_out",
    )(z, st, g, b)


def kernel(x, edge_index, edge_attr, Wq, Wk, Wv, We, O_h_W, O_h_b, O_e_W,
           O_e_b, g1h, b1h, g1e, b1e, Fh1_W, Fh1_b, Fh2_W, Fh2_b,
           Fe1_W, Fe1_b, Fe2_W, Fe2_b, g2h, b2h, g2e, b2e):
    N, D = x.shape
    E = edge_attr.shape[0]
    r = lambda v: v.reshape(1, -1)

    src = edge_index[0]
    dst = edge_index[1]
    xs, xd = _sc_gather(x, src, dst)
    cu, y1, st1e = _edge_attn(
        edge_attr, xs, xd, Wq, Wk, Wv, We, O_e_W, r(O_e_b))
    nd = _tc_segsum(cu, dst.reshape(1, E), N)
    h0, st1h = _h_proj(x, nd, O_h_W, r(O_h_b))
    zh, st2h = _ffn(h0, st1h, r(g1h), r(b1h), Fh1_W, r(Fh1_b), Fh2_W,
                    r(Fh2_b), rows=N, T=1000)
    h = _norm(zh, st2h, r(g2h), r(b2h), rows=N, T=1000)
    ze, st2e = _ffn(y1, st1e, r(g1e), r(b1e), Fe1_W, r(Fe1_b), Fe2_W,
                    r(Fe2_b), rows=E, T=512)
    e = _norm(ze, st2e, r(g2e), r(b2e), rows=E, T=512)
    return h, e


# speedup vs baseline: 10.2269x; 1.0139x over previous
import functools

import jax
import jax.numpy as jnp
from jax import lax
from jax.experimental import pallas as pl
from jax.experimental.pallas import tpu as pltpu
from jax.experimental.pallas import tpu_sc as plsc

H = 8
C = 16
EPS = 1e-5
NC = 2
NS = 16
NW = NC * NS


def _headsum_mat(dtype=jnp.float32):
    r = lax.broadcasted_iota(jnp.int32, (128, 8), 0)
    c = lax.broadcasted_iota(jnp.int32, (128, 8), 1)
    return (r // C == c).astype(dtype)


def _headexp_mat(dtype=jnp.float32):
    r = lax.broadcasted_iota(jnp.int32, (8, 128), 0)
    c = lax.broadcasted_iota(jnp.int32, (8, 128), 1)
    return (c // C == r).astype(dtype)


CH = 128


def _sc_gather(x, src, dst):
    N, D = x.shape
    E = src.shape[0]
    nchunk = E // CH
    nfull = (nchunk // NW) * NW
    nit = nchunk // NW
    nleft = nchunk - nfull
    mesh = plsc.VectorSubcoreMesh(core_axis_name="c", subcore_axis_name="s")

    @functools.partial(
        pl.kernel,
        out_type=(jax.ShapeDtypeStruct((E, D), jnp.float32),
                  jax.ShapeDtypeStruct((E, D), jnp.float32)),
        mesh=mesh,
        scratch_types=[pltpu.VMEM((CH,), jnp.int32),
                       pltpu.VMEM((CH, D), jnp.float32)],
    )
    def k(x_hbm, src_hbm, dst_hbm, xs_hbm, xd_hbm, idx_v, rows_v):
        wid = lax.axis_index("s") * NC + lax.axis_index("c")

        def one(base):
            pltpu.sync_copy(src_hbm.at[pl.ds(base, CH)], idx_v)
            pltpu.sync_copy(x_hbm.at[idx_v], rows_v)
            pltpu.sync_copy(rows_v, xs_hbm.at[pl.ds(base, CH)])
            pltpu.sync_copy(dst_hbm.at[pl.ds(base, CH)], idx_v)
            pltpu.sync_copy(x_hbm.at[idx_v], rows_v)
            pltpu.sync_copy(rows_v, xd_hbm.at[pl.ds(base, CH)])

        @pl.loop(0, nit)
        def _(j):
            one((j * NW + wid) * CH)

        if nleft:
            @pl.when(wid < nleft)
            def _():
                one((nfull + wid) * CH)

    return k(x, src, dst)


def _tc_segsum(cu, dst1, N):
    E = cu.shape[0]
    T = 2048
    G = E // T

    def body(dst_s, cu_ref, nd_ref, a0, a1, a2, a3):
        i = pl.program_id(0)
        banks = (a0, a1, a2, a3)

        @pl.when(i == 0)
        def _():
            for a in banks:
                a[...] = jnp.zeros_like(a)

        def one(k, _):
            for b, a in enumerate(banks):
                d = dst_s[0, k * 4 + b]
                a[pl.ds(d, 1), :] += cu_ref[pl.ds(k * 4 + b, 1), :]
            return 0

        lax.fori_loop(0, T // 4, one, 0, unroll=8)

        @pl.when(i == G - 1)
        def _():
            nd_ref[...] = (a0[...] + a1[...]) + (a2[...] + a3[...])

    return pl.pallas_call(
        body,
        grid=(G,),
        in_specs=[pl.BlockSpec((1, T), lambda i: (0, i),
                               memory_space=pltpu.SMEM),
                  pl.BlockSpec((T, 136), lambda i: (i, 0))],
        out_specs=pl.BlockSpec((N, 136), lambda i: (0, 0)),
        out_shape=jax.ShapeDtypeStruct((N, 136), jnp.float32),
        scratch_shapes=[pltpu.VMEM((N, 136), jnp.float32)] * 4,
        name="segsum",
    )(dst1, cu)


def _edge_attn(ea, xs, xd, Wq, Wk, Wv, We, OeW, Oeb):
    E, D = ea.shape
    T = 512
    G = E // T

    def body(ea_ref, xs_ref, xd_ref, wq, wk, wv, we, ow, ob,
             cu_ref, y1_ref, st_ref):
        i = pl.program_id(0)
        eav = ea_ref[...]
        xsv = xs_ref[...]
        P = lax.Precision.HIGHEST
        qd = jnp.dot(xd_ref[...], wq[...], precision=P)
        ks = jnp.dot(xsv, wk[...], precision=P)
        vs = jnp.dot(xsv, wv[...], precision=P)
        ep = jnp.dot(eav, we[...], precision=P)
        kj = ks + ep
        vj = vs + ep
        sf = qd * kj * 0.25
        score = jnp.dot(sf, _headsum_mat(), precision=lax.Precision.HIGHEST)
        ex = jnp.exp(score)
        exb = jnp.dot(ex, _headexp_mat(), precision=lax.Precision.HIGHEST)
        contrib = exb * vj
        cu_ref[...] = jnp.concatenate([contrib, ex], axis=1)
        y1 = eav + jnp.dot(sf, ow[...], precision=P) + ob[...]
        y1_ref[...] = y1
        s1 = jnp.sum(y1, axis=0, keepdims=True)
        s2 = jnp.sum(y1 * y1, axis=0, keepdims=True)
        upd = jnp.concatenate([s1, s2, jnp.zeros((6, 128), jnp.float32)],
                              axis=0)

        @pl.when(i == 0)
        def _():
            st_ref[...] = jnp.zeros_like(st_ref)

        st_ref[...] += upd

    full = pl.BlockSpec((128, 128), lambda i: (0, 0))
    row = pl.BlockSpec((1, 128), lambda i: (0, 0))
    tile = pl.BlockSpec((T, 128), lambda i: (i, 0))
    return pl.pallas_call(
        body,
        grid=(G,),
        in_specs=[tile, tile, tile, full, full, full, full, full, row],
        out_specs=[pl.BlockSpec((T, 136), lambda i: (i, 0)), tile,
                   pl.BlockSpec((8, 128), lambda i: (0, 0))],
        out_shape=(jax.ShapeDtypeStruct((E, 136), jnp.float32),
                   jax.ShapeDtypeStruct((E, D), jnp.float32),
                   jax.ShapeDtypeStruct((8, 128), jnp.float32)),
        name="edge_attn",
    )(ea, xs, xd, Wq, Wk, Wv, We, OeW, Oeb)


def _h_proj(x, nd, OhW, Ohb):
    N, D = x.shape
    T = 1000
    G = N // T

    def body(x_ref, nd_ref, ow, ob, h0_ref, st_ref):
        i = pl.program_id(0)
        nd = nd_ref[...]
        num = nd[:, 0:128]
        denb = jnp.dot(nd[:, 128:136], _headexp_mat(),
                       precision=lax.Precision.HIGHEST) + 1e-16
        hagg = num / denb
        h0 = jnp.dot(hagg, ow[...], precision=lax.Precision.HIGHEST) + ob[...] + x_ref[...]
        h0_ref[...] = h0
        s1 = jnp.sum(h0, axis=0, keepdims=True)
        s2 = jnp.sum(h0 * h0, axis=0, keepdims=True)
        upd = jnp.concatenate([s1, s2, jnp.zeros((6, 128), jnp.float32)],
                              axis=0)

        @pl.when(i == 0)
        def _():
            st_ref[...] = jnp.zeros_like(st_ref)

        st_ref[...] += upd

    return pl.pallas_call(
        body,
        grid=(G,),
        in_specs=[pl.BlockSpec((T, 128), lambda i: (i, 0)),
                  pl.BlockSpec((T, 136), lambda i: (i, 0)),
                  pl.BlockSpec((128, 128), lambda i: (0, 0)),
                  pl.BlockSpec((1, 128), lambda i: (0, 0))],
        out_specs=[pl.BlockSpec((T, 128), lambda i: (i, 0)),
                   pl.BlockSpec((8, 128), lambda i: (0, 0))],
        out_shape=(jax.ShapeDtypeStruct((N, D), jnp.float32),
                   jax.ShapeDtypeStruct((8, 128), jnp.float32)),
        name="h_proj",
    )(x, nd, OhW, Ohb)


def _ffn(t, st, g, b, F1W, F1b, F2W, F2b, rows, T):
    R, D = t.shape
    G = R // T
    inv = 1.0 / float(rows)

    def body(t_ref, sti_ref, gr, br, f1, f1b, f2, f2b, z_ref, st2_ref):
        i = pl.program_id(0)
        s = sti_ref[...]
        mu = s[0:1] * inv
        var = s[1:2] * inv - mu * mu
        rs = lax.rsqrt(var + EPS)
        tn = (t_ref[...] - mu) * rs * gr[...] + br[...]
        P = lax.Precision.HIGHEST
        hid = jnp.maximum(jnp.dot(tn, f1[...], precision=P) + f1b[...], 0.0)
        z = tn + jnp.dot(hid, f2[...], precision=P) + f2b[...]
        z_ref[...] = z
        s1 = jnp.sum(z, axis=0, keepdims=True)
        s2 = jnp.sum(z * z, axis=0, keepdims=True)
        upd = jnp.concatenate([s1, s2, jnp.zeros((6, 128), jnp.float32)],
                              axis=0)

        @pl.when(i == 0)
        def _():
            st2_ref[...] = jnp.zeros_like(st2_ref)

        st2_ref[...] += upd

    return pl.pallas_call(
        body,
        grid=(G,),
        in_specs=[pl.BlockSpec((T, 128), lambda i: (i, 0)),
                  pl.BlockSpec((8, 128), lambda i: (0, 0)),
                  pl.BlockSpec((1, 128), lambda i: (0, 0)),
                  pl.BlockSpec((1, 128), lambda i: (0, 0)),
                  pl.BlockSpec((128, 256), lambda i: (0, 0)),
                  pl.BlockSpec((1, 256), lambda i: (0, 0)),
                  pl.BlockSpec((256, 128), lambda i: (0, 0)),
                  pl.BlockSpec((1, 128), lambda i: (0, 0))],
        out_specs=[pl.BlockSpec((T, 128), lambda i: (i, 0)),
                   pl.BlockSpec((8, 128), lambda i: (0, 0))],
        out_shape=(jax.ShapeDtypeStruct((R, D), jnp.float32),
                   jax.ShapeDtypeStruct((8, 128), jnp.float32)),
        name="bn_ffn",
    )(t, st, g, b, F1W, F1b, F2W, F2b)


def _norm(z, st, g, b, rows, T):
    R, D = z.shape
    G = R // T
    inv = 1.0 / float(rows)

    def body(z_ref, sti_ref, gr, br, o_ref):
        s = sti_ref[...]
        mu = s[0:1] * inv
        var = s[1:2] * inv - mu * mu
        rs = lax.rsqrt(var + EPS)
        o_ref[...] = (z_ref[...] - mu) * rs * gr[...] + br[...]

    return pl.pallas_call(
        body,
        grid=(G,),
        in_specs=[pl.BlockSpec((T, 128), lambda i: (i, 0)),
                  pl.BlockSpec((8, 128), lambda i: (0, 0)),
                  pl.BlockSpec((1, 128), lambda i: (0, 0)),
                  pl.BlockSpec((1, 128), lambda i: (0, 0))],
        out_specs=pl.BlockSpec((T, 128), lambda i: (i, 0)),
        out_shape=jax.ShapeDtypeStruct((R, D), jnp.float32),
        name="bn_out",
    )(z, st, g, b)


def kernel(x, edge_index, edge_attr, Wq, Wk, Wv, We, O_h_W, O_h_b, O_e_W,
           O_e_b, g1h, b1h, g1e, b1e, Fh1_W, Fh1_b, Fh2_W, Fh2_b,
           Fe1_W, Fe1_b, Fe2_W, Fe2_b, g2h, b2h, g2e, b2e):
    N, D = x.shape
    E = edge_attr.shape[0]
    r = lambda v: v.reshape(1, -1)

    src = edge_index[0]
    dst = edge_index[1]
    xs, xd = _sc_gather(x, src, dst)
    cu, y1, st1e = _edge_attn(
        edge_attr, xs, xd, Wq, Wk, Wv, We, O_e_W, r(O_e_b))
    nd = _tc_segsum(cu, dst.reshape(1, E), N)
    h0, st1h = _h_proj(x, nd, O_h_W, r(O_h_b))
    zh, st2h = _ffn(h0, st1h, r(g1h), r(b1h), Fh1_W, r(Fh1_b), Fh2_W,
                    r(Fh2_b), rows=N, T=1000)
    h = _norm(zh, st2h, r(g2h), r(b2h), rows=N, T=1000)
    ze, st2e = _ffn(y1, st1e, r(g1e), r(b1e), Fe1_W, r(Fe1_b), Fe2_W,
                    r(Fe2_b), rows=E, T=512)
    e = _norm(ze, st2e, r(g2e), r(b2e), rows=E, T=512)
    return h, e
